# baseline (speedup 1.0000x reference)
import functools

import jax
import jax.numpy as jnp
from jax import lax
from jax.experimental import pallas as pl
from jax.experimental.pallas import tpu as pltpu
from jax.experimental.pallas import tpu_sc as plsc

EMBED_DIM = 64
CHUNK = 128


@functools.lru_cache(maxsize=None)
def _build_lookup(n_rows: int, vocab: int, embed_dim: int):
    info = plsc.get_sparse_core_info()
    num_workers = info.num_cores * info.num_subcores
    rows_per_worker = n_rows // num_workers
    n_chunks = rows_per_worker // CHUNK
    assert rows_per_worker % CHUNK == 0 and n_rows % num_workers == 0

    mesh = plsc.VectorSubcoreMesh(core_axis_name="c", subcore_axis_name="s")

    @functools.partial(
        pl.kernel,
        mesh=mesh,
        compiler_params=pltpu.CompilerParams(use_tc_tiling_on_sc=False),
        out_type=jax.ShapeDtypeStruct((n_rows, embed_dim), jnp.float32),
        scratch_types=[
            pltpu.VMEM((n_chunks, CHUNK), jnp.int32),
            pltpu.VMEM((CHUNK, embed_dim), jnp.float32),
            pltpu.SemaphoreType.DMA,
        ],
    )
    def lookup(idx_hbm, table_hbm, out_hbm, idx_v, rows_v, sem):
        wid = lax.axis_index("s") * info.num_cores + lax.axis_index("c")
        row0 = wid * n_chunks
        pltpu.sync_copy(idx_hbm.at[pl.ds(row0, n_chunks)], idx_v)

        def body(j, carry):
            pltpu.async_copy(table_hbm.at[idx_v.at[j]], rows_v, sem).wait()
            pltpu.sync_copy(rows_v, out_hbm.at[pl.ds((row0 + j) * CHUNK, CHUNK)])
            return carry

        lax.fori_loop(0, n_chunks, body, 0)

    return lookup


def kernel(tokens, token_embed_weight):
    batch, hist = tokens.shape
    vocab, embed_dim = token_embed_weight.shape
    flat = tokens.reshape(-1).astype(jnp.int32)
    idx2d = flat.reshape(-1, CHUNK)
    lookup = _build_lookup(flat.shape[0], vocab, embed_dim)
    out = lookup(idx2d, token_embed_weight)
    return out.reshape(batch, hist, embed_dim)


# speedup vs baseline: 1.1105x; 1.1105x over previous
import functools

import jax
import jax.numpy as jnp
from jax import lax
from jax.experimental import pallas as pl
from jax.experimental.pallas import tpu as pltpu
from jax.experimental.pallas import tpu_sc as plsc

EMBED_DIM = 64
CHUNK = 128
NBUF = 4


@functools.lru_cache(maxsize=None)
def _build_lookup(n_rows: int, vocab: int, embed_dim: int):
    info = plsc.get_sparse_core_info()
    num_workers = info.num_cores * info.num_subcores
    rows_per_worker = n_rows // num_workers
    n_chunks = rows_per_worker // CHUNK
    n_groups = n_chunks // NBUF
    assert rows_per_worker % CHUNK == 0 and n_rows % num_workers == 0
    assert n_chunks % NBUF == 0

    mesh = plsc.VectorSubcoreMesh(core_axis_name="c", subcore_axis_name="s")

    @functools.partial(
        pl.kernel,
        mesh=mesh,
        compiler_params=pltpu.CompilerParams(use_tc_tiling_on_sc=False),
        out_type=jax.ShapeDtypeStruct((n_rows, embed_dim), jnp.float32),
        scratch_types=[
            pltpu.VMEM((n_chunks, CHUNK), jnp.int32),
            pltpu.VMEM((NBUF, CHUNK, embed_dim), jnp.float32),
        ]
        + [pltpu.SemaphoreType.DMA] * (2 * NBUF),
    )
    def lookup(idx_hbm, table_hbm, out_hbm, idx_v, rows_v, *sems):
        sem_g = sems[:NBUF]
        sem_w = sems[NBUF:]
        wid = lax.axis_index("s") * info.num_cores + lax.axis_index("c")
        row0 = wid * n_chunks
        pltpu.sync_copy(idx_hbm.at[pl.ds(row0, n_chunks)], idx_v)

        def start_gather(j, b):
            pltpu.async_copy(table_hbm.at[idx_v.at[j]], rows_v.at[b], sem_g[b])

        def start_write(j, b):
            pltpu.async_copy(
                rows_v.at[b], out_hbm.at[pl.ds((row0 + j) * CHUNK, CHUNK)], sem_w[b]
            )

        def drain(sem):
            pltpu.make_async_copy(
                out_hbm.at[pl.ds(0, CHUNK)], rows_v.at[0], sem
            ).wait()

        for b in range(NBUF):
            start_gather(b, b)

        def group(g, carry):
            for b in range(NBUF):
                drain(sem_g[b])
                start_write(g * NBUF + b, b)
            for b in range(NBUF):
                drain(sem_w[b])
                start_gather((g + 1) * NBUF + b, b)
            return carry

        lax.fori_loop(0, n_groups - 1, group, 0)

        for b in range(NBUF):
            drain(sem_g[b])
            start_write((n_groups - 1) * NBUF + b, b)
        for b in range(NBUF):
            drain(sem_w[b])

    return lookup


def kernel(tokens, token_embed_weight):
    batch, hist = tokens.shape
    vocab, embed_dim = token_embed_weight.shape
    flat = tokens.reshape(-1).astype(jnp.int32)
    idx2d = flat.reshape(-1, CHUNK)
    lookup = _build_lookup(flat.shape[0], vocab, embed_dim)
    out = lookup(idx2d, token_embed_weight)
    return out.reshape(batch, hist, embed_dim)


# speedup vs baseline: 1.1148x; 1.0039x over previous
import functools

import jax
import jax.numpy as jnp
from jax import lax
from jax.experimental import pallas as pl
from jax.experimental.pallas import tpu as pltpu
from jax.experimental.pallas import tpu_sc as plsc

NBUF = 4
SPLIT = (128, 72)


@functools.lru_cache(maxsize=None)
def _build_lookup(batch: int, hist: int, vocab: int, embed_dim: int):
    info = plsc.get_sparse_core_info()
    num_workers = info.num_cores * info.num_subcores
    rows_per_worker = batch // num_workers
    n_groups = rows_per_worker // NBUF
    assert batch % num_workers == 0 and rows_per_worker % NBUF == 0
    assert sum(SPLIT) == hist

    mesh = plsc.VectorSubcoreMesh(core_axis_name="c", subcore_axis_name="s")

    @functools.partial(
        pl.kernel,
        mesh=mesh,
        compiler_params=pltpu.CompilerParams(use_tc_tiling_on_sc=False),
        out_type=jax.ShapeDtypeStruct((batch, hist, embed_dim), jnp.float32),
        scratch_types=[
            pltpu.VMEM((rows_per_worker, hist), jnp.int32),
            pltpu.VMEM((NBUF, hist, embed_dim), jnp.float32),
        ]
        + [pltpu.SemaphoreType.DMA] * (2 * NBUF),
    )
    def lookup(tok_hbm, table_hbm, out_hbm, idx_v, rows_v, *sems):
        sem_g = sems[:NBUF]
        sem_w = sems[NBUF:]
        wid = lax.axis_index("s") * info.num_cores + lax.axis_index("c")
        b0 = wid * rows_per_worker
        pltpu.sync_copy(tok_hbm.at[pl.ds(b0, rows_per_worker)], idx_v)

        def start_gather(r, buf):
            off = 0
            for width in SPLIT:
                pltpu.async_copy(
                    table_hbm.at[idx_v.at[r, pl.ds(off, width)]],
                    rows_v.at[buf, pl.ds(off, width)],
                    sem_g[buf],
                )
                off += width

        def start_write(r, buf):
            pltpu.async_copy(rows_v.at[buf], out_hbm.at[b0 + r], sem_w[buf])

        def drain(sem):
            pltpu.make_async_copy(out_hbm.at[0], rows_v.at[0], sem).wait()

        for b in range(NBUF):
            start_gather(b, b)

        def group(g, carry):
            for b in range(NBUF):
                drain(sem_g[b])
                start_write(g * NBUF + b, b)
            for b in range(NBUF):
                drain(sem_w[b])
                start_gather((g + 1) * NBUF + b, b)
            return carry

        lax.fori_loop(0, n_groups - 1, group, 0)

        for b in range(NBUF):
            drain(sem_g[b])
            start_write((n_groups - 1) * NBUF + b, b)
        for b in range(NBUF):
            drain(sem_w[b])

    return lookup


def kernel(tokens, token_embed_weight):
    batch, hist = tokens.shape
    vocab, embed_dim = token_embed_weight.shape
    lookup = _build_lookup(batch, hist, vocab, embed_dim)
    return lookup(tokens.astype(jnp.int32), token_embed_weight)
